# baseline (speedup 1.0000x reference)
import functools

import jax
import jax.numpy as jnp
from jax import lax
from jax.experimental import pallas as pl
from jax.experimental.pallas import tpu as pltpu
from jax.experimental.pallas import tpu_sc as plsc

LANES = 16
NB = 2
PIECE = 80


def _positionals(max_len, d_model):
    pos = jnp.arange(max_len, dtype=jnp.float32)[:, None]
    i = jnp.arange(0, d_model, 2, dtype=jnp.float32)
    div = jnp.exp(-jnp.log(10000.0) * i / d_model)
    pe = jnp.zeros((max_len, d_model), dtype=jnp.float32)
    pe = pe.at[:, 0::2].set(jnp.sin(pos * div))
    pe = pe.at[:, 1::2].set(jnp.cos(pos * div))
    return pe


def kernel(x, table):
    B, L = x.shape
    V, D = table.shape
    N = B * L
    info = plsc.get_sparse_core_info()
    NC, NS = info.num_cores, info.num_subcores
    NW = NC * NS
    per_w = N // NW
    CH = NB * L
    n_chunks = per_w // CH
    n_piece = CH // PIECE

    pe_rep = jnp.tile(_positionals(L, D), (NB, 1))
    x_flat = x.reshape(N)

    mesh = plsc.VectorSubcoreMesh(core_axis_name="c", subcore_axis_name="s")

    @functools.partial(
        pl.kernel,
        mesh=mesh,
        out_type=jax.ShapeDtypeStruct((N, D), jnp.float32),
        compiler_params=pltpu.CompilerParams(use_tc_tiling_on_sc=False),
        scratch_types=[
            pltpu.VMEM((CH,), jnp.int32),
            pltpu.VMEM((CH, D), jnp.float32),
            pltpu.VMEM((CH, D), jnp.float32),
            pltpu.SemaphoreType.DMA,
        ],
    )
    def k(x_hbm, table_hbm, pe_hbm, out_hbm, idx_v, rows_v, pe_v, sem):
        wid = lax.axis_index("s") * NC + lax.axis_index("c")
        pltpu.sync_copy(pe_hbm, pe_v)
        w_base = wid * per_w

        def chunk_body(c, carry):
            base = w_base + c * CH
            pltpu.sync_copy(x_hbm.at[pl.ds(base, CH)], idx_v)
            copies = [
                pltpu.async_copy(table_hbm.at[idx_v.at[pl.ds(j * PIECE, PIECE)]],
                                 rows_v.at[pl.ds(j * PIECE, PIECE)], sem)
                for j in range(n_piece)
            ]
            for cp in copies:
                cp.wait()

            def add_body(r, acc):
                for q in range(D // LANES):
                    s = pl.ds(q * LANES, LANES)
                    rows_v[r, s] = rows_v[r, s] + pe_v[r, s]
                return acc

            lax.fori_loop(0, CH, add_body, 0)
            pltpu.sync_copy(rows_v, out_hbm.at[pl.ds(base, CH)])
            return carry

        lax.fori_loop(0, n_chunks, chunk_body, 0)

    out = k(x_flat, table, pe_rep)
    return out.reshape(B, L, D)
